# Initial kernel scaffold; baseline (speedup 1.0000x reference)
import jax
import jax.numpy as jnp
from jax.experimental import pallas as pl


def kernel(x, hyperedge_index, hyperedge_weight, W, b):
    raise NotImplementedError("write your pallas kernel here")


# speedup vs baseline: 7.2282x; 7.2282x over previous
import functools

import jax
import jax.numpy as jnp
from jax import lax
from jax.experimental import pallas as pl
from jax.experimental.pallas import tpu as pltpu
from jax.experimental.pallas import tpu_sc as plsc

_N = 10000
_NNZ = 320000
_D = 128
_DH = _D // 2
_AUX = 16
_NC = 2
_NS = 16
_K = 128
_PS = _NNZ // _NS
_CH = 158
_PSP = _CH * _K
_RPS = 632
_NP = _RPS * _NS


def _sc_segment_pass(gather_aux: bool):
  mesh = plsc.VectorSubcoreMesh(core_axis_name="c", subcore_axis_name="s")
  out_type = (
      jax.ShapeDtypeStruct((_NC, _NP, _DH), jnp.float32),
      jax.ShapeDtypeStruct((_NC, _NP, _AUX), jnp.float32),
  )
  scratch = [
      pltpu.VMEM((_CH, _K), jnp.int32),
      pltpu.VMEM((_CH, _K), jnp.int32),
      pltpu.VMEM((_K, _DH), jnp.float32),
      pltpu.VMEM((_K, _AUX), jnp.float32),
      pltpu.VMEM_SHARED((_NP, _DH), jnp.float32),
      pltpu.VMEM_SHARED((_NP, _AUX), jnp.float32),
      pltpu.SemaphoreType.DMA,
      pltpu.SemaphoreType.DMA,
  ]

  @functools.partial(
      pl.kernel, out_type=out_type, mesh=mesh, scratch_types=scratch,
      compiler_params=pltpu.CompilerParams(use_tc_tiling_on_sc=False))
  def run(tbl0, tbl1, auxtbl, gidx, sidx, zmain, zaux, onesbuf,
          out_main, out_aux,
          gv, sv, rowbuf, auxbuf, accm, acca, sem_m, sem_a):
    c = lax.axis_index("c")
    s = lax.axis_index("s")
    pltpu.sync_copy(zmain, accm.at[pl.ds(s * _RPS, _RPS)])
    pltpu.sync_copy(zaux, acca.at[pl.ds(s * _RPS, _RPS)])
    pltpu.sync_copy(gidx.at[s], gv)
    pltpu.sync_copy(sidx.at[s], sv)
    if not gather_aux:
      pltpu.sync_copy(onesbuf, auxbuf)
    plsc.subcore_barrier()

    def make_body(tbl):
      def body(j, carry):
        g = gv.at[j]
        pltpu.async_copy(tbl.at[g], rowbuf, sem_m).wait()
        sj = sv.at[j]
        pltpu.sync_copy(rowbuf, accm.at[sj], add=True)

        @pl.when(lax.rem(j, 2) == c)
        def _aux():
          if gather_aux:
            pltpu.async_copy(auxtbl.at[g], auxbuf, sem_a).wait()
          pltpu.sync_copy(auxbuf, acca.at[sj], add=True)

        return carry
      return body

    @pl.when(c == 0)
    def _core0():
      lax.fori_loop(0, _CH, make_body(tbl0), 0)

    @pl.when(c == 1)
    def _core1():
      lax.fori_loop(0, _CH, make_body(tbl1), 0)

    plsc.subcore_barrier()
    sl = pl.ds(s * _RPS, _RPS)
    pltpu.sync_copy(accm.at[sl], out_main.at[c, sl])
    pltpu.sync_copy(acca.at[sl], out_aux.at[c, sl])

  return run


_pass_edge = _sc_segment_pass(gather_aux=False)
_pass_node = _sc_segment_pass(gather_aux=True)

_BLK = 1000
_GRID = _N // _BLK


@functools.partial(
    pl.pallas_call,
    grid=(_GRID,),
    in_specs=[
        pl.BlockSpec((_BLK, _D), lambda i: (i, 0)),
        pl.BlockSpec((_D, _D), lambda i: (0, 0)),
    ],
    out_specs=[
        pl.BlockSpec((_BLK, _DH), lambda i: (i, 0)),
        pl.BlockSpec((_BLK, _DH), lambda i: (i, 0)),
    ],
    out_shape=[
        jax.ShapeDtypeStruct((_N, _DH), jnp.float32),
        jax.ShapeDtypeStruct((_N, _DH), jnp.float32),
    ],
)
def _matmul(x_ref, w_ref, o0_ref, o1_ref):
  xl = jnp.dot(x_ref[...], w_ref[...], preferred_element_type=jnp.float32)
  o0_ref[...] = xl[:, :_DH]
  o1_ref[...] = xl[:, _DH:]


@functools.partial(
    pl.pallas_call,
    grid=(_GRID,),
    in_specs=[
        pl.BlockSpec((_NC, _BLK, _DH), lambda i: (0, i, 0)),
        pl.BlockSpec((_NC, _BLK, _AUX), lambda i: (0, i, 0)),
        pl.BlockSpec((_BLK, 1), lambda i: (i, 0)),
    ],
    out_specs=[
        pl.BlockSpec((_BLK, _DH), lambda i: (i, 0)),
        pl.BlockSpec((_BLK, _DH), lambda i: (i, 0)),
        pl.BlockSpec((_BLK, _AUX), lambda i: (i, 0)),
    ],
    out_shape=[
        jax.ShapeDtypeStruct((_N, _DH), jnp.float32),
        jax.ShapeDtypeStruct((_N, _DH), jnp.float32),
        jax.ShapeDtypeStruct((_N, _AUX), jnp.float32),
    ],
)
def _combine_mid(pe_ref, pa_ref, w_ref, tn0_ref, tn1_ref, ta_ref):
  bdeg = pa_ref[0, :, 0:1] + pa_ref[1, :, 0:1]
  binv = jnp.where(bdeg > 0, 1.0 / bdeg, 0.0)
  tn0_ref[...] = binv * pe_ref[0]
  tn1_ref[...] = binv * pe_ref[1]
  ta_ref[...] = jnp.broadcast_to(w_ref[...], (_BLK, _AUX))


@functools.partial(
    pl.pallas_call,
    grid=(_GRID,),
    in_specs=[
        pl.BlockSpec((_NC, _BLK, _DH), lambda i: (0, i, 0)),
        pl.BlockSpec((_NC, _BLK, _AUX), lambda i: (0, i, 0)),
        pl.BlockSpec((1, _D), lambda i: (0, 0)),
    ],
    out_specs=pl.BlockSpec((_BLK, _D), lambda i: (i, 0)),
    out_shape=jax.ShapeDtypeStruct((_N, _D), jnp.float32),
)
def _combine_out(pn_ref, pa_ref, b_ref, o_ref):
  deg = pa_ref[0, :, 0:1] + pa_ref[1, :, 0:1]
  dinv = jnp.where(deg > 0, 1.0 / deg, 0.0)
  full = jnp.concatenate([dinv * pn_ref[0], dinv * pn_ref[1]], axis=1)
  o_ref[...] = jnp.maximum(full + b_ref[...], 0.0)


def _pad_idx(idx, fill):
  per_s = idx.reshape(_NS, _PS)
  padded = jnp.pad(per_s, ((0, 0), (0, _PSP - _PS)), constant_values=fill)
  return padded.reshape(_NS, _CH, _K)


def kernel(x, hyperedge_index, hyperedge_weight, W, b):
  x = x.astype(jnp.float32)
  node_idx = hyperedge_index[0].astype(jnp.int32)
  edge_idx = hyperedge_index[1].astype(jnp.int32)
  node_g = _pad_idx(node_idx, 0)
  node_s = _pad_idx(node_idx, _N)
  edge_g = _pad_idx(edge_idx, 0)
  edge_s = _pad_idx(edge_idx, _N)

  xl0, xl1 = _matmul(x, W.astype(jnp.float32))

  zmain = jnp.zeros((_RPS, _DH), jnp.float32)
  zaux = jnp.zeros((_RPS, _AUX), jnp.float32)
  ones = jnp.ones((_K, _AUX), jnp.float32)
  dummy_aux = jnp.zeros((8, _AUX), jnp.float32)

  pe_main, pe_aux = _pass_edge(xl0, xl1, dummy_aux, node_g, edge_s,
                               zmain, zaux, ones)
  w2 = hyperedge_weight.astype(jnp.float32).reshape(_N, 1)
  tn0, tn1, ta = _combine_mid(pe_main, pe_aux, w2)
  pn_main, pn_aux = _pass_node(tn0, tn1, ta, edge_g, node_s,
                               zmain, zaux, ones)
  return _combine_out(pn_main, pn_aux, b.astype(jnp.float32).reshape(1, _D))


# speedup vs baseline: 9.5717x; 1.3242x over previous
import functools

import jax
import jax.numpy as jnp
from jax import lax
from jax.experimental import pallas as pl
from jax.experimental.pallas import tpu as pltpu
from jax.experimental.pallas import tpu_sc as plsc

_N = 10000
_NNZ = 320000
_D = 128
_DH = _D // 2
_AUX = 16
_NC = 2
_NS = 16
_K = 128
_PS = _NNZ // _NS
_CH = 158
_CHA = _CH + 2
_PSP = _CHA * _K
_RPS = 632
_NP = _RPS * _NS


def _sc_segment_pass(gather_aux: bool):
  mesh = plsc.VectorSubcoreMesh(core_axis_name="c", subcore_axis_name="s")
  out_type = (
      jax.ShapeDtypeStruct((_NC, _NP, _DH), jnp.float32),
      jax.ShapeDtypeStruct((_NC, _NP, _AUX), jnp.float32),
  )
  scratch = [
      pltpu.VMEM((_CHA, _K), jnp.int32),
      pltpu.VMEM((_CHA, _K), jnp.int32),
      pltpu.VMEM((_K, _DH), jnp.float32),
      pltpu.VMEM((_K, _DH), jnp.float32),
      pltpu.VMEM((_K, _AUX), jnp.float32),
      pltpu.VMEM_SHARED((_NP, _DH), jnp.float32),
      pltpu.VMEM_SHARED((_NP, _AUX), jnp.float32),
      pltpu.SemaphoreType.DMA,
      pltpu.SemaphoreType.DMA,
      pltpu.SemaphoreType.DMA,
  ]

  @functools.partial(
      pl.kernel, out_type=out_type, mesh=mesh, scratch_types=scratch,
      compiler_params=pltpu.CompilerParams(use_tc_tiling_on_sc=False))
  def run(tbl0, tbl1, auxtbl, gidx, sidx, zmain, zaux, onesbuf,
          out_main, out_aux,
          gv, sv, bufa, bufb, xbuf, accm, acca, sem_a, sem_b, sem_x):
    c = lax.axis_index("c")
    s = lax.axis_index("s")
    pltpu.sync_copy(zmain, accm.at[pl.ds(s * _RPS, _RPS)])
    pltpu.sync_copy(zaux, acca.at[pl.ds(s * _RPS, _RPS)])
    pltpu.sync_copy(gidx.at[s], gv)
    pltpu.sync_copy(sidx.at[s], sv)
    if not gather_aux:
      pltpu.sync_copy(onesbuf, xbuf)
    plsc.subcore_barrier()

    def run_loop(tbl, aux_off):
      pltpu.async_copy(tbl.at[gv.at[0]], bufa, sem_a)
      if gather_aux:
        pltpu.async_copy(auxtbl.at[gv.at[aux_off]], xbuf, sem_x)

      def body(t, carry):
        j0 = 2 * t
        j1 = j0 + 1
        ja = j0 + aux_off
        pltpu.async_copy(tbl.at[gv.at[j1]], bufb, sem_b)
        pltpu.make_async_copy(tbl.at[gv.at[j0]], bufa, sem_a).wait()
        pltpu.sync_copy(bufa, accm.at[sv.at[j0]], add=True)
        pltpu.async_copy(tbl.at[gv.at[j0 + 2]], bufa, sem_a)
        if gather_aux:
          pltpu.make_async_copy(auxtbl.at[gv.at[ja]], xbuf, sem_x).wait()
        pltpu.sync_copy(xbuf, acca.at[sv.at[ja]], add=True)
        if gather_aux:
          pltpu.async_copy(auxtbl.at[gv.at[ja + 2]], xbuf, sem_x)
        pltpu.make_async_copy(tbl.at[gv.at[j1]], bufb, sem_b).wait()
        pltpu.sync_copy(bufb, accm.at[sv.at[j1]], add=True)
        return carry

      lax.fori_loop(0, _CH // 2, body, 0)
      pltpu.make_async_copy(tbl.at[gv.at[0]], bufa, sem_a).wait()
      if gather_aux:
        pltpu.make_async_copy(auxtbl.at[gv.at[0]], xbuf, sem_x).wait()

    @pl.when(c == 0)
    def _core0():
      run_loop(tbl0, 0)

    @pl.when(c == 1)
    def _core1():
      run_loop(tbl1, 1)

    plsc.subcore_barrier()
    sl = pl.ds(s * _RPS, _RPS)
    pltpu.sync_copy(accm.at[sl], out_main.at[c, sl])
    pltpu.sync_copy(acca.at[sl], out_aux.at[c, sl])

  return run


_pass_edge = _sc_segment_pass(gather_aux=False)
_pass_node = _sc_segment_pass(gather_aux=True)

_BLK = 1000
_GRID = _N // _BLK


@functools.partial(
    pl.pallas_call,
    grid=(_GRID,),
    in_specs=[
        pl.BlockSpec((_BLK, _D), lambda i: (i, 0)),
        pl.BlockSpec((_D, _D), lambda i: (0, 0)),
    ],
    out_specs=[
        pl.BlockSpec((_BLK, _DH), lambda i: (i, 0)),
        pl.BlockSpec((_BLK, _DH), lambda i: (i, 0)),
    ],
    out_shape=[
        jax.ShapeDtypeStruct((_N, _DH), jnp.float32),
        jax.ShapeDtypeStruct((_N, _DH), jnp.float32),
    ],
)
def _matmul(x_ref, w_ref, o0_ref, o1_ref):
  xl = jnp.dot(x_ref[...], w_ref[...], preferred_element_type=jnp.float32)
  o0_ref[...] = xl[:, :_DH]
  o1_ref[...] = xl[:, _DH:]


@functools.partial(
    pl.pallas_call,
    grid=(_GRID,),
    in_specs=[
        pl.BlockSpec((_NC, _BLK, _DH), lambda i: (0, i, 0)),
        pl.BlockSpec((_NC, _BLK, _AUX), lambda i: (0, i, 0)),
        pl.BlockSpec((_BLK, 1), lambda i: (i, 0)),
    ],
    out_specs=[
        pl.BlockSpec((_BLK, _DH), lambda i: (i, 0)),
        pl.BlockSpec((_BLK, _DH), lambda i: (i, 0)),
        pl.BlockSpec((_BLK, _AUX), lambda i: (i, 0)),
    ],
    out_shape=[
        jax.ShapeDtypeStruct((_N, _DH), jnp.float32),
        jax.ShapeDtypeStruct((_N, _DH), jnp.float32),
        jax.ShapeDtypeStruct((_N, _AUX), jnp.float32),
    ],
)
def _combine_mid(pe_ref, pa_ref, w_ref, tn0_ref, tn1_ref, ta_ref):
  bdeg = pa_ref[0, :, 0:1] + pa_ref[1, :, 0:1]
  binv = jnp.where(bdeg > 0, 1.0 / bdeg, 0.0)
  tn0_ref[...] = binv * pe_ref[0]
  tn1_ref[...] = binv * pe_ref[1]
  ta_ref[...] = jnp.broadcast_to(w_ref[...], (_BLK, _AUX))


@functools.partial(
    pl.pallas_call,
    grid=(_GRID,),
    in_specs=[
        pl.BlockSpec((_NC, _BLK, _DH), lambda i: (0, i, 0)),
        pl.BlockSpec((_NC, _BLK, _AUX), lambda i: (0, i, 0)),
        pl.BlockSpec((1, _D), lambda i: (0, 0)),
    ],
    out_specs=pl.BlockSpec((_BLK, _D), lambda i: (i, 0)),
    out_shape=jax.ShapeDtypeStruct((_N, _D), jnp.float32),
)
def _combine_out(pn_ref, pa_ref, b_ref, o_ref):
  deg = pa_ref[0, :, 0:1] + pa_ref[1, :, 0:1]
  dinv = jnp.where(deg > 0, 1.0 / deg, 0.0)
  full = jnp.concatenate([dinv * pn_ref[0], dinv * pn_ref[1]], axis=1)
  o_ref[...] = jnp.maximum(full + b_ref[...], 0.0)


def _pad_idx(idx, fill):
  per_s = idx.reshape(_NS, _PS)
  padded = jnp.pad(per_s, ((0, 0), (0, _PSP - _PS)), constant_values=fill)
  return padded.reshape(_NS, _CHA, _K)


def kernel(x, hyperedge_index, hyperedge_weight, W, b):
  x = x.astype(jnp.float32)
  node_idx = hyperedge_index[0].astype(jnp.int32)
  edge_idx = hyperedge_index[1].astype(jnp.int32)
  node_g = _pad_idx(node_idx, 0)
  node_s = _pad_idx(node_idx, _N)
  edge_g = _pad_idx(edge_idx, 0)
  edge_s = _pad_idx(edge_idx, _N)

  xl0, xl1 = _matmul(x, W.astype(jnp.float32))

  zmain = jnp.zeros((_RPS, _DH), jnp.float32)
  zaux = jnp.zeros((_RPS, _AUX), jnp.float32)
  ones = jnp.ones((_K, _AUX), jnp.float32)
  dummy_aux = jnp.zeros((8, _AUX), jnp.float32)

  pe_main, pe_aux = _pass_edge(xl0, xl1, dummy_aux, node_g, edge_s,
                               zmain, zaux, ones)
  w2 = hyperedge_weight.astype(jnp.float32).reshape(_N, 1)
  tn0, tn1, ta = _combine_mid(pe_main, pe_aux, w2)
  pn_main, pn_aux = _pass_node(tn0, tn1, ta, edge_g, node_s,
                               zmain, zaux, ones)
  return _combine_out(pn_main, pn_aux, b.astype(jnp.float32).reshape(1, _D))
